# baseline (speedup 1.0000x reference)
import functools

import jax
import jax.numpy as jnp
from jax import lax
from jax.experimental import pallas as pl
from jax.experimental.pallas import tpu as pltpu
from jax.experimental.pallas import tpu_sc as plsc

_THRESHOLD = 0.1

_B = 4
_S = 4096
_D = 4096
_PER_B = _S * _D
_HALF = _PER_B // 2
_NW = 32
_PER_TILE = _HALF // _NW
_CH = 32768
_CH_PER_B = _PER_TILE // _CH
_NCH = _B * _CH_PER_B
_N = _B * _PER_B


def _mask_chunk(buf):
    @plsc.parallel_loop(0, _CH, 16, unroll=8)
    def _m(i):
        v = buf[pl.ds(i, 16)]
        buf[pl.ds(i, 16)] = jnp.where(jnp.abs(v) > _THRESHOLD, v, 0.0)


def _sc_body(x_hbm, o_hbm, buf0, buf1, isem0, isem1, osem0, osem1, csem):
    wid = lax.axis_index("s") * 2 + lax.axis_index("c")
    bufs = (buf0, buf1)
    isems = (isem0, isem1)
    osems = (osem0, osem1)

    def moff(g):
        b, j = divmod(g, _CH_PER_B)
        off = b * _PER_B + _HALF + wid * _PER_TILE + j * _CH
        return pl.multiple_of(off, _CH)

    copies = []
    for b in range(_B):
        off = pl.multiple_of(b * _PER_B + wid * _PER_TILE, _CH)
        cp = pltpu.make_async_copy(
            x_hbm.at[pl.ds(off, _PER_TILE)],
            o_hbm.at[pl.ds(off, _PER_TILE)],
            csem,
        )
        cp.start()
        copies.append(cp)

    dmas_in = [None] * _NCH
    dmas_out = [None] * _NCH

    def start_in(g):
        slot = g % 2
        dmas_in[g] = pltpu.make_async_copy(
            x_hbm.at[pl.ds(moff(g), _CH)], bufs[slot], isems[slot]
        )
        dmas_in[g].start()

    def start_out(g):
        slot = g % 2
        dmas_out[g] = pltpu.make_async_copy(
            bufs[slot], o_hbm.at[pl.ds(moff(g), _CH)], osems[slot]
        )
        dmas_out[g].start()

    for cp in copies:
        cp.wait()


_sc_kernel = functools.partial(
    pl.kernel,
    out_type=jax.ShapeDtypeStruct((_N,), jnp.float32),
    mesh=plsc.VectorSubcoreMesh(core_axis_name="c", subcore_axis_name="s"),
    scratch_types=[
        pltpu.VMEM((_CH,), jnp.float32),
        pltpu.VMEM((_CH,), jnp.float32),
        pltpu.SemaphoreType.DMA,
        pltpu.SemaphoreType.DMA,
        pltpu.SemaphoreType.DMA,
        pltpu.SemaphoreType.DMA,
        pltpu.SemaphoreType.DMA,
    ],
)(_sc_body)


def kernel(x):
    return _sc_kernel(x.reshape(-1)).reshape(x.shape)


# speedup vs baseline: 22.1060x; 22.1060x over previous
import functools

import jax
import jax.numpy as jnp
from jax import lax
from jax.experimental import pallas as pl
from jax.experimental.pallas import tpu as pltpu
from jax.experimental.pallas import tpu_sc as plsc

_THRESHOLD = 0.1

_B = 4
_S = 4096
_D = 4096
_HALF_ROWS = _S // 2
_NW = 32
_BAND = _HALF_ROWS // _NW
_CR = 8
_CC = 1024
_RC = _BAND // _CR
_CCN = _D // _CC
_PER_HALF = _B * _RC * _CCN
_NCH = 2 * _PER_HALF
_NB = 8
_PREF = 4


def _mask_chunk(buf):
    for r in range(_CR):
        @plsc.parallel_loop(0, _CC, 16, unroll=8)
        def _m(i):
            v = buf[r, pl.ds(i, 16)]
            buf[r, pl.ds(i, 16)] = jnp.where(jnp.abs(v) > _THRESHOLD, v, 0.0)


def _sc_body(x_hbm, o_hbm, *scratch):
    bufs = scratch[:_NB]
    isems = scratch[_NB:2 * _NB]
    osems = scratch[2 * _NB:3 * _NB]
    wid = lax.axis_index("s") * 2 + lax.axis_index("c")

    def addr(h):
        c = h // 2
        m = h % 2
        b = c // (_RC * _CCN)
        rc = (c // _CCN) % _RC
        cc = c % _CCN
        row = m * _HALF_ROWS + wid * _BAND + rc * _CR
        return b, pl.multiple_of(row, _CR), cc * _CC

    def in_dma(h, slot):
        b, row, col = addr(h)
        return pltpu.make_async_copy(
            x_hbm.at[b, pl.ds(row, _CR), pl.ds(col, _CC)],
            bufs[slot],
            isems[slot],
        )

    def out_dma(h, slot):
        b, row, col = addr(h)
        return pltpu.make_async_copy(
            bufs[slot],
            o_hbm.at[b, pl.ds(row, _CR), pl.ds(col, _CC)],
            osems[slot],
        )

    for s in range(_PREF):
        in_dma(s, s).start()

    def step(k, _):
        for s in range(_NB):
            h = k * _NB + s
            if s >= _PREF:
                out_dma(h - _PREF, (s - _PREF) % _NB).wait()
            else:
                @pl.when(k > 0)
                def _w():
                    out_dma(h - _PREF, (s - _PREF) % _NB).wait()

            @pl.when(h + _PREF < _NCH)
            def _p():
                in_dma(h + _PREF, (s + _PREF) % _NB).start()

            in_dma(h, s).wait()
            if s % 2 == 1:
                _mask_chunk(bufs[s])
            out_dma(h, s).start()
        return _

    lax.fori_loop(0, _NCH // _NB, step, 0)

    for h in range(_NCH - _PREF, _NCH):
        out_dma(h, h % _NB).wait()


_sc_kernel = functools.partial(
    pl.kernel,
    out_type=jax.ShapeDtypeStruct((_B, _S, _D), jnp.float32),
    mesh=plsc.VectorSubcoreMesh(core_axis_name="c", subcore_axis_name="s"),
    scratch_types=(
        [pltpu.VMEM((_CR, _CC), jnp.float32)] * _NB
        + [pltpu.SemaphoreType.DMA] * (2 * _NB)
    ),
    compiler_params=pltpu.CompilerParams(use_tc_tiling_on_sc=True),
)(_sc_body)


def kernel(x):
    return _sc_kernel(x)
